# baseline (speedup 1.0000x reference)
import functools
import jax
import jax.numpy as jnp
from jax import lax
from jax.experimental import pallas as pl
from jax.experimental.pallas import tpu as pltpu
from jax.experimental.pallas import tpu_sc as plsc

_C = 100000
_ROWS = 1024
_NL = 16
_CH = 400
_TOT = _ROWS * _C // _NL
_PERW_START = None

_info = plsc.get_sparse_core_info()
_NC, _NS = _info.num_cores, _info.num_subcores
_NW = _NC * _NS
_RPW = _TOT // _NW
_NCH = _RPW // _CH

_mesh = plsc.VectorSubcoreMesh(core_axis_name="c", subcore_axis_name="s")


@functools.partial(
    pl.kernel,
    mesh=_mesh,
    out_type=jax.ShapeDtypeStruct((_NW, _NL), jnp.float32),
    scratch_types=[
        pltpu.VMEM((_CH, _NL), jnp.float32),
        pltpu.VMEM((_CH, _NL), jnp.float32),
        pltpu.SemaphoreType.DMA,
        pltpu.SemaphoreType.DMA,
    ],
)
def _probe(x16_hbm, out_hbm, buf0, buf1, sem0, sem1):
    wid = lax.axis_index("s") * _NC + lax.axis_index("c")
    base = wid * _RPW
    bufs = (buf0, buf1)
    sems = (sem0, sem1)
    handles = [None, None]
    for i in range(_NCH):
        k = i % 2
        if handles[k] is not None:
            handles[k].wait()
        handles[k] = pltpu.async_copy(
            x16_hbm.at[pl.ds(base + i * _CH, _CH)], bufs[k], sems[k]
        )
    handles[0].wait()
    handles[1].wait()
    pltpu.sync_copy(buf0.at[0], out_hbm.at[wid])


def kernel(x, target):
    out = _probe(x.reshape(_TOT, _NL))
    return jnp.sum(out)


# speedup vs baseline: 3.7857x; 3.7857x over previous
import functools
import jax
import jax.numpy as jnp
from jax import lax
from jax.experimental import pallas as pl
from jax.experimental.pallas import tpu as pltpu
from jax.experimental.pallas import tpu_sc as plsc

_C = 100000
_SMOOTHING = 0.1
_CONF = 1.0 - _SMOOTHING
_UNI = _SMOOTHING / (_C - 1)
_ROWS = 1024
_BC = 3072
_NBLK = (_C + _BC - 1) // _BC

_NL = 16
_SW = 128
_NSLIV = _ROWS * _C // _SW


def _make_sc_gather():
    info = plsc.get_sparse_core_info()
    nc, ns = info.num_cores, info.num_subcores
    nw = nc * ns
    bpw = _ROWS // nw
    mesh = plsc.VectorSubcoreMesh(core_axis_name="c", subcore_axis_name="s")

    @functools.partial(
        pl.kernel,
        mesh=mesh,
        out_type=jax.ShapeDtypeStruct((_ROWS,), jnp.float32),
        compiler_params=pltpu.CompilerParams(needs_layout_passes=False),
        scratch_types=[
            pltpu.VMEM((bpw,), jnp.int32),
            pltpu.VMEM((bpw,), jnp.int32),
            pltpu.VMEM((bpw, _SW), jnp.float32),
            pltpu.VMEM((bpw,), jnp.float32),
            pltpu.SemaphoreType.DMA,
        ],
    )
    def gather_sc(x16_hbm, t_hbm, out_hbm, t_v, row_v, rows_v, xt_v, sem):
        wid = lax.axis_index("s") * nc + lax.axis_index("c")
        base = wid * bpw
        pltpu.sync_copy(t_hbm.at[pl.ds(base, bpw)], t_v)
        for k in range(bpw // _NL):
            t16 = t_v[pl.ds(k * _NL, _NL)]
            r16 = lax.iota(jnp.int32, _NL) + (base + k * _NL)
            flat = r16 * _C + t16
            row_v[pl.ds(k * _NL, _NL)] = flat >> 7
        pltpu.async_copy(x16_hbm.at[row_v], rows_v, sem).wait()
        for k in range(bpw // _NL):
            t16 = t_v[pl.ds(k * _NL, _NL)]
            r16 = lax.iota(jnp.int32, _NL) + (base + k * _NL)
            flat = r16 * _C + t16
            ridx = lax.iota(jnp.int32, _NL) + k * _NL
            xt_v[pl.ds(k * _NL, _NL)] = plsc.load_gather(
                rows_v, [ridx, flat & 127]
            )
        pltpu.sync_copy(xt_v, out_hbm.at[pl.ds(base, bpw)])

    return gather_sc


_gather_sc = _make_sc_gather()


def _loss_body(x_ref, xt_ref, o_ref, m_ref, s_ref, xsum_ref):
    j = pl.program_id(0)

    @pl.when(j == 0)
    def _():
        m_ref[...] = jnp.full((_ROWS, 1), -jnp.inf, jnp.float32)
        s_ref[...] = jnp.zeros((_ROWS, 1), jnp.float32)
        xsum_ref[...] = jnp.zeros((_ROWS, 1), jnp.float32)

    raw = x_ref[...]

    def accumulate(blk_ninf, blk_zero):
        bm = jnp.max(blk_ninf, axis=1, keepdims=True)
        m_old = m_ref[...]
        m_new = jnp.maximum(m_old, bm)
        s_ref[...] = s_ref[...] * jnp.exp(m_old - m_new) + jnp.sum(
            jnp.exp(blk_ninf - m_new), axis=1, keepdims=True
        )
        m_ref[...] = m_new
        xsum_ref[...] += jnp.sum(blk_zero, axis=1, keepdims=True)

    @pl.when(j < _NBLK - 1)
    def _():
        accumulate(raw, raw)

    @pl.when(j == _NBLK - 1)
    def _():
        col = jax.lax.broadcasted_iota(jnp.int32, raw.shape, 1) + j * _BC
        valid = col < _C
        accumulate(jnp.where(valid, raw, -jnp.inf), jnp.where(valid, raw, 0.0))

        logz = m_ref[...] + jnp.log(s_ref[...])
        sum_logp = xsum_ref[...] - _C * logz
        logp_t = xt_ref[...] - logz
        loss_rows = -(_UNI * sum_logp + (_CONF - _UNI) * logp_t)
        o_ref[...] = jnp.sum(loss_rows, axis=(0, 1), keepdims=True) / _ROWS


def kernel(x, target):
    xt = _gather_sc(x.reshape(_NSLIV, _SW), target.astype(jnp.int32))
    out = pl.pallas_call(
        _loss_body,
        grid=(_NBLK,),
        in_specs=[
            pl.BlockSpec((_ROWS, _BC), lambda j: (0, j)),
            pl.BlockSpec((_ROWS, 1), lambda j: (0, 0)),
        ],
        out_specs=pl.BlockSpec((1, 1), lambda j: (0, 0)),
        out_shape=jax.ShapeDtypeStruct((1, 1), jnp.float32),
        scratch_shapes=[pltpu.VMEM((_ROWS, 1), jnp.float32) for _ in range(3)],
        compiler_params=pltpu.CompilerParams(
            dimension_semantics=("arbitrary",),
        ),
    )(x, xt.reshape(_ROWS, 1))
    return out[0, 0]


# speedup vs baseline: 7.6883x; 2.0309x over previous
import jax
import jax.numpy as jnp
from jax.experimental import pallas as pl
from jax.experimental.pallas import tpu as pltpu

_C = 100000
_SMOOTHING = 0.1
_CONF = 1.0 - _SMOOTHING
_UNI = _SMOOTHING / (_C - 1)
_ROWS = 1024
_BC = 3584
_NBLK = (_C + _BC - 1) // _BC


def _loss_body(x_ref, t_ref, o_ref, m_ref, s_ref, xsum_ref, xt_ref):
    j = pl.program_id(0)

    @pl.when(j == 0)
    def _():
        m_ref[...] = jnp.full((_ROWS, 1), -jnp.inf, jnp.float32)
        s_ref[...] = jnp.zeros((_ROWS, 1), jnp.float32)
        xsum_ref[...] = jnp.zeros((_ROWS, 1), jnp.float32)
        xt_ref[...] = jnp.zeros((_ROWS, 1), jnp.float32)

    raw = x_ref[...]
    col = jax.lax.broadcasted_iota(jnp.int32, raw.shape, 1) + j * _BC

    def accumulate(blk_ninf, blk_zero):
        bm = jnp.max(blk_ninf, axis=1, keepdims=True)
        m_old = m_ref[...]
        m_new = jnp.maximum(m_old, bm)
        s_ref[...] = s_ref[...] * jnp.exp(m_old - m_new) + jnp.sum(
            jnp.exp(blk_ninf - m_new), axis=1, keepdims=True
        )
        m_ref[...] = m_new
        xsum_ref[...] += jnp.sum(blk_zero, axis=1, keepdims=True)
        xt_ref[...] += jnp.sum(
            jnp.where(col == t_ref[...], blk_zero, 0.0), axis=1, keepdims=True
        )

    @pl.when(j < _NBLK - 1)
    def _():
        accumulate(raw, raw)

    @pl.when(j == _NBLK - 1)
    def _():
        valid = col < _C
        accumulate(jnp.where(valid, raw, -jnp.inf), jnp.where(valid, raw, 0.0))

        logz = m_ref[...] + jnp.log(s_ref[...])
        sum_logp = xsum_ref[...] - _C * logz
        logp_t = xt_ref[...] - logz
        loss_rows = -(_UNI * sum_logp + (_CONF - _UNI) * logp_t)
        o_ref[...] = jnp.sum(loss_rows, axis=(0, 1), keepdims=True) / _ROWS


def kernel(x, target):
    t2d = target.astype(jnp.int32).reshape(_ROWS, 1)
    out = pl.pallas_call(
        _loss_body,
        grid=(_NBLK,),
        in_specs=[
            pl.BlockSpec((_ROWS, _BC), lambda j: (0, j)),
            pl.BlockSpec((_ROWS, 1), lambda j: (0, 0)),
        ],
        out_specs=pl.BlockSpec((1, 1), lambda j: (0, 0)),
        out_shape=jax.ShapeDtypeStruct((1, 1), jnp.float32),
        scratch_shapes=[pltpu.VMEM((_ROWS, 1), jnp.float32) for _ in range(4)],
        compiler_params=pltpu.CompilerParams(
            dimension_semantics=("arbitrary",),
            vmem_limit_bytes=64 * 1024 * 1024,
        ),
    )(x, t2d)
    return out[0, 0]


# speedup vs baseline: 7.8084x; 1.0156x over previous
import jax
import jax.numpy as jnp
from jax.experimental import pallas as pl
from jax.experimental.pallas import tpu as pltpu

_C = 100000
_SMOOTHING = 0.1
_CONF = 1.0 - _SMOOTHING
_UNI = _SMOOTHING / (_C - 1)
_ROWS = 1024
_BC = 3072
_NBLK = (_C + _BC - 1) // _BC


def _loss_body(x_ref, t_ref, col_ref, o_ref, m_ref, s_ref, xsum_ref, xt_ref):
    j = pl.program_id(0)

    @pl.when(j == 0)
    def _():
        m_ref[...] = jnp.full((_ROWS, 1), -jnp.inf, jnp.float32)
        s_ref[...] = jnp.zeros((_ROWS, 1), jnp.float32)
        xsum_ref[...] = jnp.zeros((_ROWS, 1), jnp.float32)
        xt_ref[...] = jnp.zeros((_ROWS, 1), jnp.float32)

    raw = x_ref[...]
    col = col_ref[...]

    def accumulate(blk_ninf, blk_zero):
        bm = jnp.max(blk_ninf, axis=1, keepdims=True)
        m_old = m_ref[...]
        m_new = jnp.maximum(m_old, bm)
        s_ref[...] = s_ref[...] * jnp.exp(m_old - m_new) + jnp.sum(
            jnp.exp(blk_ninf - m_new), axis=1, keepdims=True
        )
        m_ref[...] = m_new
        xsum_ref[...] += jnp.sum(blk_zero, axis=1, keepdims=True)
        xt_ref[...] += jnp.sum(
            jnp.where(col == t_ref[...], blk_zero, 0.0), axis=1, keepdims=True
        )

    @pl.when(j < _NBLK - 1)
    def _():
        accumulate(raw, raw)

    @pl.when(j == _NBLK - 1)
    def _():
        valid = col < _C
        accumulate(jnp.where(valid, raw, -jnp.inf), jnp.where(valid, raw, 0.0))

        logz = m_ref[...] + jnp.log(s_ref[...])
        sum_logp = xsum_ref[...] - _C * logz
        logp_t = xt_ref[...] - logz
        loss_rows = -(_UNI * sum_logp + (_CONF - _UNI) * logp_t)
        o_ref[...] = jnp.sum(loss_rows, axis=(0, 1), keepdims=True) / _ROWS


def kernel(x, target):
    t2d = target.astype(jnp.int32).reshape(_ROWS, 1)
    cols = jnp.arange(_NBLK * _BC, dtype=jnp.int32).reshape(1, _NBLK * _BC)
    out = pl.pallas_call(
        _loss_body,
        grid=(_NBLK,),
        in_specs=[
            pl.BlockSpec((_ROWS, _BC), lambda j: (0, j)),
            pl.BlockSpec((_ROWS, 1), lambda j: (0, 0)),
            pl.BlockSpec((1, _BC), lambda j: (0, j)),
        ],
        out_specs=pl.BlockSpec((1, 1), lambda j: (0, 0)),
        out_shape=jax.ShapeDtypeStruct((1, 1), jnp.float32),
        scratch_shapes=[pltpu.VMEM((_ROWS, 1), jnp.float32) for _ in range(4)],
        compiler_params=pltpu.CompilerParams(
            dimension_semantics=("arbitrary",),
        ),
    )(x, t2d, cols)
    return out[0, 0]
